# Initial kernel scaffold; baseline (speedup 1.0000x reference)
import jax
import jax.numpy as jnp
from jax.experimental import pallas as pl


def kernel(x, W1, b1, W2, b2, W3, b3, We1, be1, We2, be2):
    raise NotImplementedError("write your pallas kernel here")


# speedup vs baseline: 1.0434x; 1.0434x over previous
import functools
import jax
import jax.numpy as jnp
from jax.experimental import pallas as pl
from jax.experimental.pallas import tpu as pltpu


def _mm_bias_kernel(a_ref, w_ref, b_ref, o_ref, *, relu):
    acc = jnp.dot(a_ref[...], w_ref[...], preferred_element_type=jnp.float32)
    acc = acc + b_ref[...]
    if relu:
        acc = jnp.maximum(acc, 0.0)
    o_ref[...] = acc


def _mm_bias(a, w, b, relu, bm, bn):
    M, K = a.shape
    K2, N = w.shape
    grid = (N // bn, M // bm)
    return pl.pallas_call(
        functools.partial(_mm_bias_kernel, relu=relu),
        grid=grid,
        in_specs=[
            pl.BlockSpec((bm, K), lambda n, m: (m, 0)),
            pl.BlockSpec((K, bn), lambda n, m: (0, n)),
            pl.BlockSpec((1, bn), lambda n, m: (0, n)),
        ],
        out_specs=pl.BlockSpec((bm, bn), lambda n, m: (m, n)),
        out_shape=jax.ShapeDtypeStruct((M, N), jnp.float32),
    )(a, w, b.reshape(1, N))


def _mm_gate_kernel(a_ref, w_ref, b_ref, g_ref, o_ref):
    acc = jnp.dot(a_ref[...], w_ref[...], preferred_element_type=jnp.float32)
    acc = acc + b_ref[...]
    o_ref[...] = acc * g_ref[...]


def _mm_gate(a, w, b, gate, bm):
    M, K = a.shape
    K2, N = w.shape
    grid = (M // bm,)
    return pl.pallas_call(
        _mm_gate_kernel,
        grid=grid,
        in_specs=[
            pl.BlockSpec((bm, K), lambda m: (m, 0)),
            pl.BlockSpec((K, N), lambda m: (0, 0)),
            pl.BlockSpec((1, N), lambda m: (0, 0)),
            pl.BlockSpec((bm, 1), lambda m: (m, 0)),
        ],
        out_specs=pl.BlockSpec((bm, N), lambda m: (m, 0)),
        out_shape=jax.ShapeDtypeStruct((M, N), jnp.float32),
    )(a, w, b.reshape(1, N), gate)


def _route_kernel(l_ref, g_ref, run_ref, *, E, capacity, bt):
    i = pl.program_id(0)

    @pl.when(i == 0)
    def _():
        run_ref[...] = jnp.zeros_like(run_ref)

    logits = l_ref[...][:, :E]
    m = jnp.max(logits, axis=1, keepdims=True)
    ex = jnp.exp(logits - m)
    p = ex / jnp.sum(ex, axis=1, keepdims=True)
    iota = jax.lax.broadcasted_iota(jnp.int32, (bt, E), 1)
    m0 = jnp.max(p, axis=1, keepdims=True)
    e0 = jnp.min(jnp.where(p == m0, iota, E), axis=1, keepdims=True)
    oh0 = iota == e0
    pm = jnp.where(oh0, -jnp.inf, p)
    m1 = jnp.max(pm, axis=1, keepdims=True)
    e1 = jnp.min(jnp.where(pm == m1, iota, E), axis=1, keepdims=True)
    oh1 = iota == e1
    c = oh0.astype(jnp.float32) + oh1.astype(jnp.float32)
    r = jax.lax.broadcasted_iota(jnp.int32, (bt, bt), 0)
    cc = jax.lax.broadcasted_iota(jnp.int32, (bt, bt), 1)
    tri = (cc < r).astype(jnp.float32)
    pos = jnp.dot(tri, c, preferred_element_type=jnp.float32) + run_ref[...]
    p0 = jnp.sum(jnp.where(oh0, pos, 0.0), axis=1, keepdims=True)
    p1 = jnp.sum(jnp.where(oh1, pos, 0.0), axis=1, keepdims=True)
    keep0 = (p0 < capacity).astype(jnp.float32)
    keep1 = (p1 < capacity).astype(jnp.float32)
    g_ref[...] = m0 * keep0 + m1 * keep1
    run_ref[...] += jnp.sum(c, axis=0, keepdims=True)


def _route(logits_padded, E, capacity, bt=128):
    T = logits_padded.shape[0]
    return pl.pallas_call(
        functools.partial(_route_kernel, E=E, capacity=capacity, bt=bt),
        grid=(T // bt,),
        in_specs=[pl.BlockSpec((bt, logits_padded.shape[1]), lambda i: (i, 0))],
        out_specs=pl.BlockSpec((bt, 1), lambda i: (i, 0)),
        out_shape=jax.ShapeDtypeStruct((T, 1), jnp.float32),
        scratch_shapes=[pltpu.VMEM((1, E), jnp.float32)],
        compiler_params=pltpu.CompilerParams(
            dimension_semantics=("arbitrary",)),
    )(logits_padded)


def kernel(x, W1, b1, W2, b2, W3, b3, We1, be1, We2, be2):
    B, T, C = x.shape
    E = W3.shape[1]
    k = 2
    capacity = int(T / E * 1.25)
    xf = x.reshape(T, C)

    h1 = _mm_bias(xf, W1, b1, relu=True, bm=2048, bn=512)
    h2 = _mm_bias(h1, W2, b2, relu=True, bm=512, bn=1024)
    W3p = jnp.pad(W3, ((0, 0), (0, 128 - E)))
    b3p = jnp.pad(b3, (0, 128 - E))
    logits = _mm_bias(h2, W3p, b3p, relu=False, bm=512, bn=128)

    gate = _route(logits, E, capacity)

    y1 = _mm_bias(xf, We1, be1, relu=True, bm=2048, bn=512)
    out = _mm_gate(y1, We2, be2, gate, bm=512)
    return out.reshape(B, T, C)
